# Initial kernel scaffold; baseline (speedup 1.0000x reference)
import jax
import jax.numpy as jnp
from jax.experimental import pallas as pl


def kernel(x, edge_index, W1, b1, W2, b2):
    raise NotImplementedError("write your pallas kernel here")


# speedup vs baseline: 17.3462x; 17.3462x over previous
import functools

import jax
import jax.numpy as jnp
from jax import lax
from jax.experimental import pallas as pl
from jax.experimental.pallas import tpu as pltpu
from jax.experimental.pallas import tpu_sc as plsc

_N = 10000
_E = 320000
_D = 128
_H = 64
_C = 40
_CP = 48

_NC = 2
_NS = 16
_NW = _NC * _NS
_EPW = _E // _NW
_CH = 80
_NCHUNK = _EPW // _CH
_RPT = 624

_BM = 1000
_NBLK = _N // _BM


def _sc_mesh():
    return plsc.VectorSubcoreMesh(core_axis_name="c", subcore_axis_name="s")


def _deg_body(dst_hbm, ones_hbm, z_hbm, out0_hbm, out1_hbm, acc, ones_v, didx_v,
              stage_v):
    cid = lax.axis_index("c")
    sid = lax.axis_index("s")
    wid = sid * _NC + cid

    pltpu.sync_copy(ones_hbm, ones_v)
    base_r = sid * _RPT
    pltpu.sync_copy(z_hbm.at[pl.ds(0, _RPT)], stage_v)
    pltpu.sync_copy(stage_v, acc.at[pl.ds(base_r, _RPT)])

    @pl.when(sid == _NS - 1)
    def _():
        pltpu.sync_copy(stage_v.at[pl.ds(0, 16)], acc.at[pl.ds(_N - 16, 16)])

    plsc.subcore_barrier()

    def chunk(i, carry):
        base = pl.multiple_of(wid * _EPW + i * _CH, _CH)
        pltpu.sync_copy(dst_hbm.at[pl.ds(base, _CH)], didx_v)
        pltpu.sync_copy(ones_v, acc.at[didx_v], add=True)
        return carry

    lax.fori_loop(0, _NCHUNK, chunk, 0)
    plsc.subcore_barrier()

    pltpu.sync_copy(acc.at[pl.ds(base_r, _RPT)], stage_v)
    for c, out_hbm in ((0, out0_hbm), (1, out1_hbm)):
        @pl.when(cid == c)
        def _():
            pltpu.sync_copy(stage_v, out_hbm.at[pl.ds(base_r, _RPT)])

    @pl.when(sid == _NS - 1)
    def _():
        pltpu.sync_copy(acc.at[pl.ds(_N - 16, 16)], stage_v.at[pl.ds(0, 16)])
        for c, out_hbm in ((0, out0_hbm), (1, out1_hbm)):
            @pl.when(cid == c)
            def _():
                pltpu.sync_copy(stage_v.at[pl.ds(0, 16)], out_hbm.at[pl.ds(_N - 16, 16)])


def _deg_call(dst, ones, zeros1d):
    k = pl.kernel(
        _deg_body,
        out_type=(jax.ShapeDtypeStruct((_N,), jnp.float32),
                  jax.ShapeDtypeStruct((_N,), jnp.float32)),
        mesh=_sc_mesh(),
        compiler_params=pltpu.CompilerParams(use_tc_tiling_on_sc=False),
        scratch_types=[
            pltpu.VMEM_SHARED((_N,), jnp.float32),
            pltpu.VMEM((_CH,), jnp.float32),
            pltpu.VMEM((_CH,), jnp.int32),
            pltpu.VMEM((_RPT,), jnp.float32),
        ],
    )
    return k(dst, ones, zeros1d)


def _scat_body(src_hbm, dst_hbm, hs_hbm, z_hbm, out0_hbm, out1_hbm, acc, table,
               sidx_v, didx_v, rows_v, stage_v, sem, *, F):
    cid = lax.axis_index("c")
    sid = lax.axis_index("s")
    wid = sid * _NC + cid

    base_r = sid * _RPT
    pltpu.sync_copy(z_hbm.at[pl.ds(0, _RPT)], stage_v)
    pltpu.sync_copy(stage_v, acc.at[pl.ds(base_r, _RPT)])

    @pl.when(sid == _NS - 1)
    def _():
        pltpu.sync_copy(stage_v.at[pl.ds(0, 16)], acc.at[pl.ds(_N - 16, 16)])

    pltpu.sync_copy(hs_hbm.at[pl.ds(base_r, _RPT)], stage_v)
    pltpu.sync_copy(stage_v, table.at[pl.ds(base_r, _RPT)])

    @pl.when(sid == _NS - 1)
    def _():
        pltpu.sync_copy(hs_hbm.at[pl.ds(_N - 16, 16)], stage_v.at[pl.ds(0, 16)])
        pltpu.sync_copy(stage_v.at[pl.ds(0, 16)], table.at[pl.ds(_N - 16, 16)])

    plsc.subcore_barrier()

    def chunk(i, carry):
        base = pl.multiple_of(wid * _EPW + i * _CH, _CH)
        pltpu.sync_copy(src_hbm.at[pl.ds(base, _CH)], sidx_v)
        pltpu.sync_copy(dst_hbm.at[pl.ds(base, _CH)], didx_v)
        pltpu.async_copy(table.at[sidx_v], rows_v, sem).wait()
        pltpu.sync_copy(rows_v, acc.at[didx_v], add=True)
        return carry

    lax.fori_loop(0, _NCHUNK, chunk, 0)
    plsc.subcore_barrier()

    pltpu.sync_copy(acc.at[pl.ds(base_r, _RPT)], stage_v)
    for c, out_hbm in ((0, out0_hbm), (1, out1_hbm)):
        @pl.when(cid == c)
        def _():
            pltpu.sync_copy(stage_v, out_hbm.at[pl.ds(base_r, _RPT)])

    @pl.when(sid == _NS - 1)
    def _():
        pltpu.sync_copy(acc.at[pl.ds(_N - 16, 16)], stage_v.at[pl.ds(0, 16)])
        for c, out_hbm in ((0, out0_hbm), (1, out1_hbm)):
            @pl.when(cid == c)
            def _():
                pltpu.sync_copy(stage_v.at[pl.ds(0, 16)], out_hbm.at[pl.ds(_N - 16, 16)])


def _scat_call(src, dst, hs, zeros2d, F):
    k = pl.kernel(
        functools.partial(_scat_body, F=F),
        out_type=(jax.ShapeDtypeStruct((_N, F), jnp.float32),
                  jax.ShapeDtypeStruct((_N, F), jnp.float32)),
        mesh=_sc_mesh(),
        compiler_params=pltpu.CompilerParams(use_tc_tiling_on_sc=False),
        scratch_types=[
            pltpu.VMEM_SHARED((_N, F), jnp.float32),
            pltpu.VMEM_SHARED((_N, F), jnp.float32),
            pltpu.VMEM((_CH,), jnp.int32),
            pltpu.VMEM((_CH,), jnp.int32),
            pltpu.VMEM((_CH, F), jnp.float32),
            pltpu.VMEM((_RPT, F), jnp.float32),
            pltpu.SemaphoreType.DMA,
        ],
    )
    return k(src, dst, hs, zeros2d)


def _tc1_body(c0_ref, c1_ref, x_ref, w1_ref, h1_ref, hs1_ref, dinv_ref):
    c = c0_ref[...] + c1_ref[...]
    dinv = lax.rsqrt(c + 1.0)
    h = jnp.dot(x_ref[...], w1_ref[...], preferred_element_type=jnp.float32)
    h1_ref[...] = h
    hs1_ref[...] = h * dinv
    dinv_ref[...] = dinv


def _tc1_call(c0, c1, x, W1):
    return pl.pallas_call(
        _tc1_body,
        grid=(_NBLK,),
        in_specs=[
            pl.BlockSpec((_BM, 1), lambda i: (i, 0)),
            pl.BlockSpec((_BM, 1), lambda i: (i, 0)),
            pl.BlockSpec((_BM, _D), lambda i: (i, 0)),
            pl.BlockSpec((_D, _H), lambda i: (0, 0)),
        ],
        out_specs=[
            pl.BlockSpec((_BM, _H), lambda i: (i, 0)),
            pl.BlockSpec((_BM, _H), lambda i: (i, 0)),
            pl.BlockSpec((_BM, 1), lambda i: (i, 0)),
        ],
        out_shape=[
            jax.ShapeDtypeStruct((_N, _H), jnp.float32),
            jax.ShapeDtypeStruct((_N, _H), jnp.float32),
            jax.ShapeDtypeStruct((_N, 1), jnp.float32),
        ],
    )(c0, c1, x, W1)


def _tc2_body(n1a_ref, n1b_ref, h1_ref, dinv_ref, b1_ref, w2_ref, h2_ref, hs2_ref):
    n = n1a_ref[...] + n1b_ref[...]
    dinv = dinv_ref[...]
    out1 = jax.nn.relu(dinv * (n + dinv * h1_ref[...]) + b1_ref[...])
    h2 = jnp.dot(out1, w2_ref[...], preferred_element_type=jnp.float32)
    h2_ref[...] = h2
    hs2_ref[...] = jnp.concatenate(
        [h2 * dinv, jnp.zeros((_BM, _CP - _C), jnp.float32)], axis=1)


def _tc2_call(n1a, n1b, h1, dinv, b1r, W2):
    return pl.pallas_call(
        _tc2_body,
        grid=(_NBLK,),
        in_specs=[
            pl.BlockSpec((_BM, _H), lambda i: (i, 0)),
            pl.BlockSpec((_BM, _H), lambda i: (i, 0)),
            pl.BlockSpec((_BM, _H), lambda i: (i, 0)),
            pl.BlockSpec((_BM, 1), lambda i: (i, 0)),
            pl.BlockSpec((1, _H), lambda i: (0, 0)),
            pl.BlockSpec((_H, _C), lambda i: (0, 0)),
        ],
        out_specs=[
            pl.BlockSpec((_BM, _C), lambda i: (i, 0)),
            pl.BlockSpec((_BM, _CP), lambda i: (i, 0)),
        ],
        out_shape=[
            jax.ShapeDtypeStruct((_N, _C), jnp.float32),
            jax.ShapeDtypeStruct((_N, _CP), jnp.float32),
        ],
    )(n1a, n1b, h1, dinv, b1r, W2)


def _tc3_body(n2a_ref, n2b_ref, h2_ref, dinv_ref, b2_ref, lsm_ref, xo_ref):
    n = (n2a_ref[...] + n2b_ref[...])[:, : _C]
    dinv = dinv_ref[...]
    xo = dinv * (n + dinv * h2_ref[...]) + b2_ref[...]
    m = jnp.max(xo, axis=1, keepdims=True)
    lse = jnp.log(jnp.sum(jnp.exp(xo - m), axis=1, keepdims=True)) + m
    lsm_ref[...] = xo - lse
    xo_ref[...] = xo


def _tc3_call(n2a, n2b, h2, dinv, b2r):
    return pl.pallas_call(
        _tc3_body,
        grid=(_NBLK,),
        in_specs=[
            pl.BlockSpec((_BM, _CP), lambda i: (i, 0)),
            pl.BlockSpec((_BM, _CP), lambda i: (i, 0)),
            pl.BlockSpec((_BM, _C), lambda i: (i, 0)),
            pl.BlockSpec((_BM, 1), lambda i: (i, 0)),
            pl.BlockSpec((1, _C), lambda i: (0, 0)),
        ],
        out_specs=[
            pl.BlockSpec((_BM, _C), lambda i: (i, 0)),
            pl.BlockSpec((_BM, _C), lambda i: (i, 0)),
        ],
        out_shape=[
            jax.ShapeDtypeStruct((_N, _C), jnp.float32),
            jax.ShapeDtypeStruct((_N, _C), jnp.float32),
        ],
    )(n2a, n2b, h2, dinv, b2r)


def kernel(x, edge_index, W1, b1, W2, b2):
    ones = jnp.ones((_CH,), jnp.float32)
    zeros1d = jnp.zeros((_RPT + 16,), jnp.float32)
    zerosH = jnp.zeros((_RPT + 16, _H), jnp.float32)
    zerosP = jnp.zeros((_RPT + 16, _CP), jnp.float32)

    src = edge_index[0]
    dst = edge_index[1]
    c0, c1 = _deg_call(dst, ones, zeros1d)
    h1, hs1, dinv = _tc1_call(c0.reshape(_N, 1), c1.reshape(_N, 1), x, W1)
    n1a, n1b = _scat_call(src, dst, hs1, zerosH, _H)
    h2, hs2 = _tc2_call(n1a, n1b, h1, dinv, b1.reshape(1, _H), W2)
    n2a, n2b = _scat_call(src, dst, hs2, zerosP, _CP)
    lsm, xo = _tc3_call(n2a, n2b, h2, dinv, b2.reshape(1, _C))
    return lsm, xo


# speedup vs baseline: 35.2404x; 2.0316x over previous
import functools

import jax
import jax.numpy as jnp
from jax import lax
from jax.experimental import pallas as pl
from jax.experimental.pallas import tpu as pltpu
from jax.experimental.pallas import tpu_sc as plsc

_N = 10000
_E = 320000
_D = 128
_H = 64
_C = 40
_CP = 48

_NC = 2
_NS = 16
_NW = _NC * _NS
_EPW = _E // _NW
_CH = 40
_NCHUNK = _EPW // _CH
_NBUF = 5
_NGRP = _NCHUNK // _NBUF
_RPT = 624
_SR = 104

_BM = 1000
_NBLK = _N // _BM


def _sc_mesh():
    return plsc.VectorSubcoreMesh(core_axis_name="c", subcore_axis_name="s")


def _worker_ids():
    cid = lax.axis_index("c")
    sid = lax.axis_index("s")
    return cid, sid, sid * _NC + cid


def _deg_body(dst2_hbm, ones_hbm, z_hbm, out0_hbm, out1_hbm, acc, ones_v,
              didx2, stage_v, *sems):
    cid, sid, wid = _worker_ids()

    pltpu.sync_copy(ones_hbm, ones_v)
    pltpu.sync_copy(dst2_hbm.at[pl.ds(wid * _NCHUNK, _NCHUNK)], didx2)
    base_r = sid * _RPT
    pltpu.sync_copy(z_hbm.at[pl.ds(0, _RPT)], stage_v)
    pltpu.sync_copy(stage_v, acc.at[pl.ds(base_r, _RPT)])

    @pl.when(sid == _NS - 1)
    def _():
        pltpu.sync_copy(stage_v.at[pl.ds(0, 16)], acc.at[pl.ds(_N - 16, 16)])

    plsc.subcore_barrier()

    def start(i, b):
        pltpu.async_copy(ones_v, acc.at[didx2.at[i]], sems[b], add=True)

    def wait(b):
        pltpu.make_async_copy(ones_v, acc.at[didx2.at[0]], sems[b]).wait()

    for b in range(_NBUF):
        start(b, b)

    def group(g, carry):
        for b in range(_NBUF):
            i = g * _NBUF + b
            wait(b)
            start(i + _NBUF, b)
        return carry

    lax.fori_loop(0, _NGRP - 1, group, 0)
    for b in range(_NBUF):
        wait(b)

    plsc.subcore_barrier()

    pltpu.sync_copy(acc.at[pl.ds(base_r, _RPT)], stage_v)
    for c, out_hbm in ((0, out0_hbm), (1, out1_hbm)):
        @pl.when(cid == c)
        def _():
            pltpu.sync_copy(stage_v, out_hbm.at[pl.ds(base_r, _RPT)])

    @pl.when(sid == _NS - 1)
    def _():
        pltpu.sync_copy(acc.at[pl.ds(_N - 16, 16)], stage_v.at[pl.ds(0, 16)])
        for c, out_hbm in ((0, out0_hbm), (1, out1_hbm)):
            @pl.when(cid == c)
            def _():
                pltpu.sync_copy(stage_v.at[pl.ds(0, 16)], out_hbm.at[pl.ds(_N - 16, 16)])


def _deg_call(dst2, ones, zeros1d):
    k = pl.kernel(
        _deg_body,
        out_type=(jax.ShapeDtypeStruct((_N,), jnp.float32),
                  jax.ShapeDtypeStruct((_N,), jnp.float32)),
        mesh=_sc_mesh(),
        compiler_params=pltpu.CompilerParams(use_tc_tiling_on_sc=False),
        scratch_types=[
            pltpu.VMEM_SHARED((_N,), jnp.float32),
            pltpu.VMEM((_CH,), jnp.float32),
            pltpu.VMEM((_NCHUNK, _CH), jnp.int32),
            pltpu.VMEM((_RPT,), jnp.float32),
        ] + [pltpu.SemaphoreType.DMA] * _NBUF,
    )
    return k(dst2, ones, zeros1d)


def _scat_body(src2_hbm, dst2_hbm, hs_hbm, z_hbm, out0_hbm, out1_hbm, acc,
               table, sidx2, didx2, stage_v, *bufs_and_sems, F):
    rows = bufs_and_sems[:_NBUF]
    sems = bufs_and_sems[_NBUF:]
    cid, sid, wid = _worker_ids()

    base_r = sid * _RPT
    pltpu.sync_copy(src2_hbm.at[pl.ds(wid * _NCHUNK, _NCHUNK)], sidx2)
    pltpu.sync_copy(dst2_hbm.at[pl.ds(wid * _NCHUNK, _NCHUNK)], didx2)
    pltpu.sync_copy(z_hbm, stage_v)
    for p in range(_RPT // _SR):
        pltpu.sync_copy(stage_v, acc.at[pl.ds(base_r + p * _SR, _SR)])

    @pl.when(sid == _NS - 1)
    def _():
        pltpu.sync_copy(stage_v.at[pl.ds(0, 16)], acc.at[pl.ds(_N - 16, 16)])

    for p in range(_RPT // _SR):
        pltpu.sync_copy(hs_hbm.at[pl.ds(base_r + p * _SR, _SR)], stage_v)
        pltpu.sync_copy(stage_v, table.at[pl.ds(base_r + p * _SR, _SR)])

    @pl.when(sid == _NS - 1)
    def _():
        pltpu.sync_copy(hs_hbm.at[pl.ds(_N - 16, 16)], stage_v.at[pl.ds(0, 16)])
        pltpu.sync_copy(stage_v.at[pl.ds(0, 16)], table.at[pl.ds(_N - 16, 16)])

    plsc.subcore_barrier()

    def start_gather(i, b):
        pltpu.async_copy(table.at[sidx2.at[i]], rows[b], sems[b])

    def wait_gather(b):
        pltpu.make_async_copy(table.at[sidx2.at[0]], rows[b], sems[b]).wait()

    for b in range(_NBUF):
        start_gather(b, b)

    def group(g, carry):
        for b in range(_NBUF):
            i = g * _NBUF + b
            wait_gather(b)
            pltpu.sync_copy(rows[b], acc.at[didx2.at[i]], add=True)
            start_gather(i + _NBUF, b)
        return carry

    lax.fori_loop(0, _NGRP - 1, group, 0)
    for b in range(_NBUF):
        i = (_NGRP - 1) * _NBUF + b
        wait_gather(b)
        pltpu.sync_copy(rows[b], acc.at[didx2.at[i]], add=True)

    plsc.subcore_barrier()

    for c, out_hbm in ((0, out0_hbm), (1, out1_hbm)):
        @pl.when(cid == c)
        def _():
            for p in range(_RPT // _SR):
                pltpu.sync_copy(acc.at[pl.ds(base_r + p * _SR, _SR)], stage_v)
                pltpu.sync_copy(stage_v, out_hbm.at[pl.ds(base_r + p * _SR, _SR)])

    @pl.when(sid == _NS - 1)
    def _():
        pltpu.sync_copy(acc.at[pl.ds(_N - 16, 16)], stage_v.at[pl.ds(0, 16)])
        for c, out_hbm in ((0, out0_hbm), (1, out1_hbm)):
            @pl.when(cid == c)
            def _():
                pltpu.sync_copy(stage_v.at[pl.ds(0, 16)], out_hbm.at[pl.ds(_N - 16, 16)])


def _scat_call(src2, dst2, hs, zeros2d, F):
    k = pl.kernel(
        functools.partial(_scat_body, F=F),
        out_type=(jax.ShapeDtypeStruct((_N, F), jnp.float32),
                  jax.ShapeDtypeStruct((_N, F), jnp.float32)),
        mesh=_sc_mesh(),
        compiler_params=pltpu.CompilerParams(use_tc_tiling_on_sc=False),
        scratch_types=[
            pltpu.VMEM_SHARED((_N, F), jnp.float32),
            pltpu.VMEM_SHARED((_N, F), jnp.float32),
            pltpu.VMEM((_NCHUNK, _CH), jnp.int32),
            pltpu.VMEM((_NCHUNK, _CH), jnp.int32),
            pltpu.VMEM((_SR, F), jnp.float32),
        ] + [pltpu.VMEM((_CH, F), jnp.float32)] * _NBUF
          + [pltpu.SemaphoreType.DMA] * _NBUF,
    )
    return k(src2, dst2, hs, zeros2d)


def _tc1_body(c0_ref, c1_ref, x_ref, w1_ref, h1_ref, hs1_ref, dinv_ref):
    c = c0_ref[...] + c1_ref[...]
    dinv = lax.rsqrt(c + 1.0)
    h = jnp.dot(x_ref[...], w1_ref[...], preferred_element_type=jnp.float32)
    h1_ref[...] = h
    hs1_ref[...] = h * dinv
    dinv_ref[...] = dinv


def _tc1_call(c0, c1, x, W1):
    return pl.pallas_call(
        _tc1_body,
        grid=(_NBLK,),
        in_specs=[
            pl.BlockSpec((_BM, 1), lambda i: (i, 0)),
            pl.BlockSpec((_BM, 1), lambda i: (i, 0)),
            pl.BlockSpec((_BM, _D), lambda i: (i, 0)),
            pl.BlockSpec((_D, _H), lambda i: (0, 0)),
        ],
        out_specs=[
            pl.BlockSpec((_BM, _H), lambda i: (i, 0)),
            pl.BlockSpec((_BM, _H), lambda i: (i, 0)),
            pl.BlockSpec((_BM, 1), lambda i: (i, 0)),
        ],
        out_shape=[
            jax.ShapeDtypeStruct((_N, _H), jnp.float32),
            jax.ShapeDtypeStruct((_N, _H), jnp.float32),
            jax.ShapeDtypeStruct((_N, 1), jnp.float32),
        ],
    )(c0, c1, x, W1)


def _tc2_body(n1a_ref, n1b_ref, h1_ref, dinv_ref, b1_ref, w2_ref, h2_ref, hs2_ref):
    n = n1a_ref[...] + n1b_ref[...]
    dinv = dinv_ref[...]
    out1 = jax.nn.relu(dinv * (n + dinv * h1_ref[...]) + b1_ref[...])
    h2 = jnp.dot(out1, w2_ref[...], preferred_element_type=jnp.float32)
    h2_ref[...] = h2
    hs2_ref[...] = jnp.concatenate(
        [h2 * dinv, jnp.zeros((_BM, _CP - _C), jnp.float32)], axis=1)


def _tc2_call(n1a, n1b, h1, dinv, b1r, W2):
    return pl.pallas_call(
        _tc2_body,
        grid=(_NBLK,),
        in_specs=[
            pl.BlockSpec((_BM, _H), lambda i: (i, 0)),
            pl.BlockSpec((_BM, _H), lambda i: (i, 0)),
            pl.BlockSpec((_BM, _H), lambda i: (i, 0)),
            pl.BlockSpec((_BM, 1), lambda i: (i, 0)),
            pl.BlockSpec((1, _H), lambda i: (0, 0)),
            pl.BlockSpec((_H, _C), lambda i: (0, 0)),
        ],
        out_specs=[
            pl.BlockSpec((_BM, _C), lambda i: (i, 0)),
            pl.BlockSpec((_BM, _CP), lambda i: (i, 0)),
        ],
        out_shape=[
            jax.ShapeDtypeStruct((_N, _C), jnp.float32),
            jax.ShapeDtypeStruct((_N, _CP), jnp.float32),
        ],
    )(n1a, n1b, h1, dinv, b1r, W2)


def _tc3_body(n2a_ref, n2b_ref, h2_ref, dinv_ref, b2_ref, lsm_ref, xo_ref):
    n = (n2a_ref[...] + n2b_ref[...])[:, : _C]
    dinv = dinv_ref[...]
    xo = dinv * (n + dinv * h2_ref[...]) + b2_ref[...]
    m = jnp.max(xo, axis=1, keepdims=True)
    lse = jnp.log(jnp.sum(jnp.exp(xo - m), axis=1, keepdims=True)) + m
    lsm_ref[...] = xo - lse
    xo_ref[...] = xo


def _tc3_call(n2a, n2b, h2, dinv, b2r):
    return pl.pallas_call(
        _tc3_body,
        grid=(_NBLK,),
        in_specs=[
            pl.BlockSpec((_BM, _CP), lambda i: (i, 0)),
            pl.BlockSpec((_BM, _CP), lambda i: (i, 0)),
            pl.BlockSpec((_BM, _C), lambda i: (i, 0)),
            pl.BlockSpec((_BM, 1), lambda i: (i, 0)),
            pl.BlockSpec((1, _C), lambda i: (0, 0)),
        ],
        out_specs=[
            pl.BlockSpec((_BM, _C), lambda i: (i, 0)),
            pl.BlockSpec((_BM, _C), lambda i: (i, 0)),
        ],
        out_shape=[
            jax.ShapeDtypeStruct((_N, _C), jnp.float32),
            jax.ShapeDtypeStruct((_N, _C), jnp.float32),
        ],
    )(n2a, n2b, h2, dinv, b2r)


def kernel(x, edge_index, W1, b1, W2, b2):
    ones = jnp.ones((_CH,), jnp.float32)
    zeros1d = jnp.zeros((_RPT,), jnp.float32)
    zerosH = jnp.zeros((_SR, _H), jnp.float32)
    zerosP = jnp.zeros((_SR, _CP), jnp.float32)

    src2 = edge_index[0].reshape(_E // _CH, _CH)
    dst2 = edge_index[1].reshape(_E // _CH, _CH)
    c0, c1 = _deg_call(dst2, ones, zeros1d)
    h1, hs1, dinv = _tc1_call(c0.reshape(_N, 1), c1.reshape(_N, 1), x, W1)
    n1a, n1b = _scat_call(src2, dst2, hs1, zerosH, _H)
    h2, hs2 = _tc2_call(n1a, n1b, h1, dinv, b1.reshape(1, _H), W2)
    n2a, n2b = _scat_call(src2, dst2, hs2, zerosP, _CP)
    lsm, xo = _tc3_call(n2a, n2b, h2, dinv, b2.reshape(1, _C))
    return lsm, xo


# speedup vs baseline: 38.4150x; 1.0901x over previous
import functools

import jax
import jax.numpy as jnp
from jax import lax
from jax.experimental import pallas as pl
from jax.experimental.pallas import tpu as pltpu
from jax.experimental.pallas import tpu_sc as plsc

_N = 10000
_E = 320000
_D = 128
_H = 64
_C = 40
_CP = 48

_NC = 2
_NS = 16
_NW = _NC * _NS
_EPW = _E // _NW
_CH = 40
_NCHUNK = _EPW // _CH
_NBUF = 5
_NGRP = _NCHUNK // _NBUF
_RPT = 624
_SR = 104

_BM = 2000
_NBLK = _N // _BM


def _sc_mesh():
    return plsc.VectorSubcoreMesh(core_axis_name="c", subcore_axis_name="s")


def _worker_ids():
    cid = lax.axis_index("c")
    sid = lax.axis_index("s")
    return cid, sid, sid * _NC + cid


def _deg_body(ei_hbm, ones_hbm, z_hbm, out0_hbm, out1_hbm, acc, ones_v,
              didx, stage_v, *sems):
    cid, sid, wid = _worker_ids()

    pltpu.sync_copy(ones_hbm, ones_v)
    pltpu.sync_copy(ei_hbm.at[1, pl.ds(wid * _EPW, _EPW)], didx)
    base_r = sid * _RPT
    pltpu.sync_copy(z_hbm.at[pl.ds(0, _RPT)], stage_v)
    pltpu.sync_copy(stage_v, acc.at[pl.ds(base_r, _RPT)])

    @pl.when(sid == _NS - 1)
    def _():
        pltpu.sync_copy(stage_v.at[pl.ds(0, 16)], acc.at[pl.ds(_N - 16, 16)])

    plsc.subcore_barrier()

    def start(i, b):
        pltpu.async_copy(ones_v, acc.at[didx.at[pl.ds(i * _CH, _CH)]], sems[b],
                         add=True)

    def wait(b):
        pltpu.make_async_copy(ones_v, acc.at[didx.at[pl.ds(0, _CH)]], sems[b]).wait()

    for b in range(_NBUF):
        start(b, b)

    def group(g, carry):
        for b in range(_NBUF):
            i = g * _NBUF + b
            wait(b)
            start(i + _NBUF, b)
        return carry

    lax.fori_loop(0, _NGRP - 1, group, 0)
    for b in range(_NBUF):
        wait(b)

    plsc.subcore_barrier()

    pltpu.sync_copy(acc.at[pl.ds(base_r, _RPT)], stage_v)
    for c, out_hbm in ((0, out0_hbm), (1, out1_hbm)):
        @pl.when(cid == c)
        def _():
            pltpu.sync_copy(stage_v, out_hbm.at[pl.ds(base_r, _RPT)])

    @pl.when(sid == _NS - 1)
    def _():
        pltpu.sync_copy(acc.at[pl.ds(_N - 16, 16)], stage_v.at[pl.ds(0, 16)])
        for c, out_hbm in ((0, out0_hbm), (1, out1_hbm)):
            @pl.when(cid == c)
            def _():
                pltpu.sync_copy(stage_v.at[pl.ds(0, 16)], out_hbm.at[pl.ds(_N - 16, 16)])


def _deg_call(ei, ones, zeros1d):
    k = pl.kernel(
        _deg_body,
        out_type=(jax.ShapeDtypeStruct((_N,), jnp.float32),
                  jax.ShapeDtypeStruct((_N,), jnp.float32)),
        mesh=_sc_mesh(),
        compiler_params=pltpu.CompilerParams(use_tc_tiling_on_sc=False),
        scratch_types=[
            pltpu.VMEM_SHARED((_N,), jnp.float32),
            pltpu.VMEM((_CH,), jnp.float32),
            pltpu.VMEM((_EPW,), jnp.int32),
            pltpu.VMEM((_RPT,), jnp.float32),
        ] + [pltpu.SemaphoreType.DMA] * _NBUF,
    )
    return k(ei, ones, zeros1d)


def _scat_body(ei_hbm, hs_hbm, z_hbm, out0_hbm, out1_hbm, acc,
               table, sidx, didx, stage_v, *bufs_and_sems, F):
    rows = bufs_and_sems[:_NBUF]
    sems = bufs_and_sems[_NBUF:]
    cid, sid, wid = _worker_ids()

    base_r = sid * _RPT
    pltpu.sync_copy(ei_hbm.at[0, pl.ds(wid * _EPW, _EPW)], sidx)
    pltpu.sync_copy(ei_hbm.at[1, pl.ds(wid * _EPW, _EPW)], didx)
    pltpu.sync_copy(z_hbm, stage_v)
    for p in range(_RPT // _SR):
        pltpu.sync_copy(stage_v, acc.at[pl.ds(base_r + p * _SR, _SR)])

    @pl.when(sid == _NS - 1)
    def _():
        pltpu.sync_copy(stage_v.at[pl.ds(0, 16)], acc.at[pl.ds(_N - 16, 16)])

    for p in range(_RPT // _SR):
        pltpu.sync_copy(hs_hbm.at[pl.ds(base_r + p * _SR, _SR)], stage_v)
        pltpu.sync_copy(stage_v, table.at[pl.ds(base_r + p * _SR, _SR)])

    @pl.when(sid == _NS - 1)
    def _():
        pltpu.sync_copy(hs_hbm.at[pl.ds(_N - 16, 16)], stage_v.at[pl.ds(0, 16)])
        pltpu.sync_copy(stage_v.at[pl.ds(0, 16)], table.at[pl.ds(_N - 16, 16)])

    plsc.subcore_barrier()

    def start_gather(i, b):
        pltpu.async_copy(table.at[sidx.at[pl.ds(i * _CH, _CH)]], rows[b], sems[b])

    def wait_gather(b):
        pltpu.make_async_copy(table.at[sidx.at[pl.ds(0, _CH)]], rows[b], sems[b]).wait()

    for b in range(_NBUF):
        start_gather(b, b)

    def group(g, carry):
        for b in range(_NBUF):
            i = g * _NBUF + b
            wait_gather(b)
            pltpu.sync_copy(rows[b], acc.at[didx.at[pl.ds(i * _CH, _CH)]], add=True)
            start_gather(i + _NBUF, b)
        return carry

    lax.fori_loop(0, _NGRP - 1, group, 0)
    for b in range(_NBUF):
        i = (_NGRP - 1) * _NBUF + b
        wait_gather(b)
        pltpu.sync_copy(rows[b], acc.at[didx.at[pl.ds(i * _CH, _CH)]], add=True)

    plsc.subcore_barrier()

    for c, out_hbm in ((0, out0_hbm), (1, out1_hbm)):
        @pl.when(cid == c)
        def _():
            for p in range(_RPT // _SR):
                pltpu.sync_copy(acc.at[pl.ds(base_r + p * _SR, _SR)], stage_v)
                pltpu.sync_copy(stage_v, out_hbm.at[pl.ds(base_r + p * _SR, _SR)])

    @pl.when(sid == _NS - 1)
    def _():
        pltpu.sync_copy(acc.at[pl.ds(_N - 16, 16)], stage_v.at[pl.ds(0, 16)])
        for c, out_hbm in ((0, out0_hbm), (1, out1_hbm)):
            @pl.when(cid == c)
            def _():
                pltpu.sync_copy(stage_v.at[pl.ds(0, 16)], out_hbm.at[pl.ds(_N - 16, 16)])


def _scat_call(ei, hs, zeros2d, F):
    k = pl.kernel(
        functools.partial(_scat_body, F=F),
        out_type=(jax.ShapeDtypeStruct((_N, F), jnp.float32),
                  jax.ShapeDtypeStruct((_N, F), jnp.float32)),
        mesh=_sc_mesh(),
        compiler_params=pltpu.CompilerParams(use_tc_tiling_on_sc=False),
        scratch_types=[
            pltpu.VMEM_SHARED((_N, F), jnp.float32),
            pltpu.VMEM_SHARED((_N, F), jnp.float32),
            pltpu.VMEM((_EPW,), jnp.int32),
            pltpu.VMEM((_EPW,), jnp.int32),
            pltpu.VMEM((_SR, F), jnp.float32),
        ] + [pltpu.VMEM((_CH, F), jnp.float32)] * _NBUF
          + [pltpu.SemaphoreType.DMA] * _NBUF,
    )
    return k(ei, hs, zeros2d)


def _tc1_body(c0_ref, c1_ref, x_ref, w1_ref, h1_ref, hs1_ref, dinv_ref):
    c = c0_ref[...] + c1_ref[...]
    dinv = jnp.transpose(lax.rsqrt(c + 1.0), (1, 0))
    h = jnp.dot(x_ref[...], w1_ref[...], preferred_element_type=jnp.float32)
    h1_ref[...] = h
    hs1_ref[...] = h * dinv
    dinv_ref[...] = dinv


def _tc1_call(c0, c1, x, W1):
    return pl.pallas_call(
        _tc1_body,
        out_shape=[
            jax.ShapeDtypeStruct((_N, _H), jnp.float32),
            jax.ShapeDtypeStruct((_N, _H), jnp.float32),
            jax.ShapeDtypeStruct((_N, 1), jnp.float32),
        ],
    )(c0, c1, x, W1)


def _tc2_body(n1a_ref, n1b_ref, h1_ref, dinv_ref, b1_ref, w2_ref, h2_ref, hs2_ref):
    n = n1a_ref[...] + n1b_ref[...]
    dinv = dinv_ref[...]
    out1 = jax.nn.relu(dinv * (n + dinv * h1_ref[...]) + b1_ref[...])
    h2 = jnp.dot(out1, w2_ref[...], preferred_element_type=jnp.float32)
    h2_ref[...] = h2
    hs2_ref[...] = jnp.concatenate(
        [h2 * dinv, jnp.zeros((_N, _CP - _C), jnp.float32)], axis=1)


def _tc2_call(n1a, n1b, h1, dinv, b1r, W2):
    return pl.pallas_call(
        _tc2_body,
        out_shape=[
            jax.ShapeDtypeStruct((_N, _C), jnp.float32),
            jax.ShapeDtypeStruct((_N, _CP), jnp.float32),
        ],
    )(n1a, n1b, h1, dinv, b1r, W2)


def _tc3_body(n2a_ref, n2b_ref, h2_ref, dinv_ref, b2_ref, lsm_ref, xo_ref):
    n = (n2a_ref[...] + n2b_ref[...])[:, : _C]
    dinv = dinv_ref[...]
    xo = dinv * (n + dinv * h2_ref[...]) + b2_ref[...]
    m = jnp.max(xo, axis=1, keepdims=True)
    lse = jnp.log(jnp.sum(jnp.exp(xo - m), axis=1, keepdims=True)) + m
    lsm_ref[...] = xo - lse
    xo_ref[...] = xo


def _tc3_call(n2a, n2b, h2, dinv, b2r):
    return pl.pallas_call(
        _tc3_body,
        out_shape=[
            jax.ShapeDtypeStruct((_N, _C), jnp.float32),
            jax.ShapeDtypeStruct((_N, _C), jnp.float32),
        ],
    )(n2a, n2b, h2, dinv, b2r)


def kernel(x, edge_index, W1, b1, W2, b2):
    ones = jnp.ones((_CH,), jnp.float32)
    zeros1d = jnp.zeros((_RPT,), jnp.float32)
    zerosH = jnp.zeros((_SR, _H), jnp.float32)
    zerosP = jnp.zeros((_SR, _CP), jnp.float32)

    c0, c1 = _deg_call(edge_index, ones, zeros1d)
    h1, hs1, dinv = _tc1_call(c0.reshape(1, _N), c1.reshape(1, _N), x, W1)
    n1a, n1b = _scat_call(edge_index, hs1, zerosH, _H)
    h2, hs2 = _tc2_call(n1a, n1b, h1, dinv, b1.reshape(1, _H), W2)
    n2a, n2b = _scat_call(edge_index, hs2, zerosP, _CP)
    lsm, xo = _tc3_call(n2a, n2b, h2, dinv, b2.reshape(1, _C))
    return lsm, xo


# speedup vs baseline: 42.6230x; 1.1095x over previous
import functools

import jax
import jax.numpy as jnp
from jax import lax
from jax.experimental import pallas as pl
from jax.experimental.pallas import tpu as pltpu
from jax.experimental.pallas import tpu_sc as plsc

_N = 10000
_E = 320000
_D = 128
_H = 64
_C = 40
_CP = 48

_NC = 2
_NS = 16
_NW = _NC * _NS
_EPW = _E // _NW
_CH = 80
_NCHUNK = _EPW // _CH
_NBUF = 5
_NGRP = _NCHUNK // _NBUF
_NBS = 4
_RPT = 624
_SR = 104

_BM = 2000
_NBLK = _N // _BM


def _sc_mesh():
    return plsc.VectorSubcoreMesh(core_axis_name="c", subcore_axis_name="s")


def _worker_ids():
    cid = lax.axis_index("c")
    sid = lax.axis_index("s")
    return cid, sid, sid * _NC + cid


def _deg_body(ei_hbm, ones_hbm, z_hbm, out0_hbm, out1_hbm, acc, ones_v,
              didx, stage_v, *sems):
    cid, sid, wid = _worker_ids()

    pltpu.sync_copy(ones_hbm, ones_v)
    pltpu.sync_copy(ei_hbm.at[1, pl.ds(wid * _EPW, _EPW)], didx)
    base_r = sid * _RPT
    pltpu.sync_copy(z_hbm.at[pl.ds(0, _RPT)], stage_v)
    pltpu.sync_copy(stage_v, acc.at[pl.ds(base_r, _RPT)])

    @pl.when(sid == _NS - 1)
    def _():
        pltpu.sync_copy(stage_v.at[pl.ds(0, 16)], acc.at[pl.ds(_N - 16, 16)])

    plsc.subcore_barrier()

    def start(i, b):
        pltpu.async_copy(ones_v, acc.at[didx.at[pl.ds(i * _CH, _CH)]], sems[b],
                         add=True)

    def wait(b):
        pltpu.make_async_copy(ones_v, acc.at[didx.at[pl.ds(0, _CH)]], sems[b]).wait()

    for b in range(_NBUF):
        start(b, b)

    def group(g, carry):
        for b in range(_NBUF):
            i = g * _NBUF + b
            wait(b)
            start(i + _NBUF, b)
        return carry

    lax.fori_loop(0, _NGRP - 1, group, 0)
    for b in range(_NBUF):
        wait(b)

    plsc.subcore_barrier()

    pltpu.sync_copy(acc.at[pl.ds(base_r, _RPT)], stage_v)
    for c, out_hbm in ((0, out0_hbm), (1, out1_hbm)):
        @pl.when(cid == c)
        def _():
            pltpu.sync_copy(stage_v, out_hbm.at[pl.ds(base_r, _RPT)])

    @pl.when(sid == _NS - 1)
    def _():
        pltpu.sync_copy(acc.at[pl.ds(_N - 16, 16)], stage_v.at[pl.ds(0, 16)])
        for c, out_hbm in ((0, out0_hbm), (1, out1_hbm)):
            @pl.when(cid == c)
            def _():
                pltpu.sync_copy(stage_v.at[pl.ds(0, 16)], out_hbm.at[pl.ds(_N - 16, 16)])


def _deg_call(ei, ones, zeros1d):
    k = pl.kernel(
        _deg_body,
        out_type=(jax.ShapeDtypeStruct((_N,), jnp.float32),
                  jax.ShapeDtypeStruct((_N,), jnp.float32)),
        mesh=_sc_mesh(),
        compiler_params=pltpu.CompilerParams(use_tc_tiling_on_sc=False),
        scratch_types=[
            pltpu.VMEM_SHARED((_N,), jnp.float32),
            pltpu.VMEM((_CH,), jnp.float32),
            pltpu.VMEM((_EPW,), jnp.int32),
            pltpu.VMEM((_RPT,), jnp.float32),
        ] + [pltpu.SemaphoreType.DMA] * _NBUF,
    )
    return k(ei, ones, zeros1d)


def _scat_body(ei_hbm, hs_hbm, z_hbm, out0_hbm, out1_hbm, acc,
               table, sidx, didx, stage_v, *bufs_and_sems, F):
    rows = bufs_and_sems[:_NBS]
    sems = bufs_and_sems[_NBS:]
    cid, sid, wid = _worker_ids()

    base_r = sid * _RPT
    pltpu.sync_copy(ei_hbm.at[0, pl.ds(wid * _EPW, _EPW)], sidx)
    pltpu.sync_copy(ei_hbm.at[1, pl.ds(wid * _EPW, _EPW)], didx)
    pltpu.sync_copy(z_hbm, stage_v)
    for p in range(_RPT // _SR):
        pltpu.sync_copy(stage_v, acc.at[pl.ds(base_r + p * _SR, _SR)])

    @pl.when(sid == _NS - 1)
    def _():
        pltpu.sync_copy(stage_v.at[pl.ds(0, 16)], acc.at[pl.ds(_N - 16, 16)])

    for p in range(_RPT // _SR):
        pltpu.sync_copy(hs_hbm.at[pl.ds(base_r + p * _SR, _SR)], stage_v)
        pltpu.sync_copy(stage_v, table.at[pl.ds(base_r + p * _SR, _SR)])

    @pl.when(sid == _NS - 1)
    def _():
        pltpu.sync_copy(hs_hbm.at[pl.ds(_N - 16, 16)], stage_v.at[pl.ds(0, 16)])
        pltpu.sync_copy(stage_v.at[pl.ds(0, 16)], table.at[pl.ds(_N - 16, 16)])

    plsc.subcore_barrier()

    gsem = sems[:_NBS]
    ssem = sems[_NBS:]

    def start_gather(i, b):
        pltpu.async_copy(table.at[sidx.at[pl.ds(i * _CH, _CH)]], rows[b], gsem[b])

    def wait_gather(b):
        pltpu.make_async_copy(table.at[sidx.at[pl.ds(0, _CH)]], rows[b], gsem[b]).wait()

    def start_scatter(i, b):
        pltpu.async_copy(rows[b], acc.at[didx.at[pl.ds(i * _CH, _CH)]], ssem[b],
                         add=True)

    def wait_scatter(b):
        pltpu.make_async_copy(rows[b], acc.at[didx.at[pl.ds(0, _CH)]], ssem[b]).wait()

    for b in range(_NBS):
        start_gather(b, b)
    for k in (0, 1):
        wait_gather(k)
        start_scatter(k, k)

    def group(g, carry):
        for r in range(_NBS):
            k = g * _NBS + 2 + r
            b = (2 + r) % _NBS
            wait_gather(b)
            start_scatter(k, b)
            wait_scatter(r)
            start_gather(k + 2, r)
        return carry

    lax.fori_loop(0, (_NCHUNK - 5) // _NBS, group, 0)
    wait_gather(2)
    start_scatter(122, 2)
    wait_scatter(0)
    start_gather(124, 0)
    wait_gather(3)
    start_scatter(123, 3)
    wait_gather(0)
    start_scatter(124, 0)
    for b in (1, 2, 3, 0):
        wait_scatter(b)

    plsc.subcore_barrier()

    for c, out_hbm in ((0, out0_hbm), (1, out1_hbm)):
        @pl.when(cid == c)
        def _():
            for p in range(_RPT // _SR):
                pltpu.sync_copy(acc.at[pl.ds(base_r + p * _SR, _SR)], stage_v)
                pltpu.sync_copy(stage_v, out_hbm.at[pl.ds(base_r + p * _SR, _SR)])

    @pl.when(sid == _NS - 1)
    def _():
        pltpu.sync_copy(acc.at[pl.ds(_N - 16, 16)], stage_v.at[pl.ds(0, 16)])
        for c, out_hbm in ((0, out0_hbm), (1, out1_hbm)):
            @pl.when(cid == c)
            def _():
                pltpu.sync_copy(stage_v.at[pl.ds(0, 16)], out_hbm.at[pl.ds(_N - 16, 16)])


def _scat_call(ei, hs, zeros2d, F):
    k = pl.kernel(
        functools.partial(_scat_body, F=F),
        out_type=(jax.ShapeDtypeStruct((_N, F), jnp.float32),
                  jax.ShapeDtypeStruct((_N, F), jnp.float32)),
        mesh=_sc_mesh(),
        compiler_params=pltpu.CompilerParams(use_tc_tiling_on_sc=False),
        scratch_types=[
            pltpu.VMEM_SHARED((_N, F), jnp.float32),
            pltpu.VMEM_SHARED((_N, F), jnp.float32),
            pltpu.VMEM((_EPW,), jnp.int32),
            pltpu.VMEM((_EPW,), jnp.int32),
            pltpu.VMEM((_SR, F), jnp.float32),
        ] + [pltpu.VMEM((_CH, F), jnp.float32)] * _NBS
          + [pltpu.SemaphoreType.DMA] * (2 * _NBS),
    )
    return k(ei, hs, zeros2d)


def _tc1_body(c0_ref, c1_ref, x_ref, w1_ref, h1_ref, hs1_ref, dinv_ref):
    c = c0_ref[...] + c1_ref[...]
    dinv = jnp.transpose(lax.rsqrt(c + 1.0), (1, 0))
    h = jnp.dot(x_ref[...], w1_ref[...], preferred_element_type=jnp.float32)
    h1_ref[...] = h
    hs1_ref[...] = h * dinv
    dinv_ref[...] = dinv


def _tc1_call(c0, c1, x, W1):
    return pl.pallas_call(
        _tc1_body,
        out_shape=[
            jax.ShapeDtypeStruct((_N, _H), jnp.float32),
            jax.ShapeDtypeStruct((_N, _H), jnp.float32),
            jax.ShapeDtypeStruct((_N, 1), jnp.float32),
        ],
    )(c0, c1, x, W1)


def _tc2_body(n1a_ref, n1b_ref, h1_ref, dinv_ref, b1_ref, w2_ref, h2_ref, hs2_ref):
    n = n1a_ref[...] + n1b_ref[...]
    dinv = dinv_ref[...]
    out1 = jax.nn.relu(dinv * (n + dinv * h1_ref[...]) + b1_ref[...])
    h2 = jnp.dot(out1, w2_ref[...], preferred_element_type=jnp.float32)
    h2_ref[...] = h2
    hs2_ref[...] = jnp.concatenate(
        [h2 * dinv, jnp.zeros((_N, _CP - _C), jnp.float32)], axis=1)


def _tc2_call(n1a, n1b, h1, dinv, b1r, W2):
    return pl.pallas_call(
        _tc2_body,
        out_shape=[
            jax.ShapeDtypeStruct((_N, _C), jnp.float32),
            jax.ShapeDtypeStruct((_N, _CP), jnp.float32),
        ],
    )(n1a, n1b, h1, dinv, b1r, W2)


def _tc3_body(n2a_ref, n2b_ref, h2_ref, dinv_ref, b2_ref, lsm_ref, xo_ref):
    n = (n2a_ref[...] + n2b_ref[...])[:, : _C]
    dinv = dinv_ref[...]
    xo = dinv * (n + dinv * h2_ref[...]) + b2_ref[...]
    m = jnp.max(xo, axis=1, keepdims=True)
    lse = jnp.log(jnp.sum(jnp.exp(xo - m), axis=1, keepdims=True)) + m
    lsm_ref[...] = xo - lse
    xo_ref[...] = xo


def _tc3_call(n2a, n2b, h2, dinv, b2r):
    return pl.pallas_call(
        _tc3_body,
        out_shape=[
            jax.ShapeDtypeStruct((_N, _C), jnp.float32),
            jax.ShapeDtypeStruct((_N, _C), jnp.float32),
        ],
    )(n2a, n2b, h2, dinv, b2r)


def kernel(x, edge_index, W1, b1, W2, b2):
    ones = jnp.ones((_CH,), jnp.float32)
    zeros1d = jnp.zeros((_RPT,), jnp.float32)
    zerosH = jnp.zeros((_SR, _H), jnp.float32)
    zerosP = jnp.zeros((_SR, _CP), jnp.float32)

    c0, c1 = _deg_call(edge_index, ones, zeros1d)
    h1, hs1, dinv = _tc1_call(c0.reshape(1, _N), c1.reshape(1, _N), x, W1)
    n1a, n1b = _scat_call(edge_index, hs1, zerosH, _H)
    h2, hs2 = _tc2_call(n1a, n1b, h1, dinv, b1.reshape(1, _H), W2)
    n2a, n2b = _scat_call(edge_index, hs2, zerosP, _CP)
    lsm, xo = _tc3_call(n2a, n2b, h2, dinv, b2.reshape(1, _C))
    return lsm, xo


# speedup vs baseline: 43.4768x; 1.0200x over previous
import functools

import jax
import jax.numpy as jnp
from jax import lax
from jax.experimental import pallas as pl
from jax.experimental.pallas import tpu as pltpu
from jax.experimental.pallas import tpu_sc as plsc

_N = 10000
_E = 320000
_D = 128
_H = 64
_C = 40
_CP = 48

_NC = 2
_NS = 16
_NW = _NC * _NS
_EPW = _E // _NW
_CH = 80
_NCHUNK = _EPW // _CH
_NBUF = 5
_NGRP = _NCHUNK // _NBUF
_NBS = 4
_RPT = 624
_SR = 104

_BM = 2000
_NBLK = _N // _BM


def _sc_mesh():
    return plsc.VectorSubcoreMesh(core_axis_name="c", subcore_axis_name="s")


def _worker_ids():
    cid = lax.axis_index("c")
    sid = lax.axis_index("s")
    return cid, sid, sid * _NC + cid


def _deg_body(ei_hbm, ones_hbm, z_hbm, out0_hbm, out1_hbm, acc, ones_v,
              didx, stage_v, *sems):
    cid, sid, wid = _worker_ids()

    pltpu.sync_copy(ones_hbm, ones_v)
    pltpu.sync_copy(ei_hbm.at[1, pl.ds(wid * _EPW, _EPW)], didx)
    base_r = sid * _RPT
    pltpu.sync_copy(z_hbm.at[pl.ds(0, _RPT)], stage_v)
    pltpu.sync_copy(stage_v, acc.at[pl.ds(base_r, _RPT)])

    @pl.when(sid == _NS - 1)
    def _():
        pltpu.sync_copy(stage_v.at[pl.ds(0, 16)], acc.at[pl.ds(_N - 16, 16)])

    plsc.subcore_barrier()

    def start(i, b):
        pltpu.async_copy(ones_v, acc.at[didx.at[pl.ds(i * _CH, _CH)]], sems[b],
                         add=True)

    def wait(b):
        pltpu.make_async_copy(ones_v, acc.at[didx.at[pl.ds(0, _CH)]], sems[b]).wait()

    for b in range(_NBUF):
        start(b, b)

    def group(g, carry):
        for b in range(_NBUF):
            i = g * _NBUF + b
            wait(b)
            start(i + _NBUF, b)
        return carry

    lax.fori_loop(0, _NGRP - 1, group, 0)
    for b in range(_NBUF):
        wait(b)

    plsc.subcore_barrier()

    pltpu.sync_copy(acc.at[pl.ds(base_r, _RPT)], stage_v)
    for c, out_hbm in ((0, out0_hbm), (1, out1_hbm)):
        @pl.when(cid == c)
        def _():
            pltpu.sync_copy(stage_v, out_hbm.at[pl.ds(base_r, _RPT)])

    @pl.when(sid == _NS - 1)
    def _():
        pltpu.sync_copy(acc.at[pl.ds(_N - 16, 16)], stage_v.at[pl.ds(0, 16)])
        for c, out_hbm in ((0, out0_hbm), (1, out1_hbm)):
            @pl.when(cid == c)
            def _():
                pltpu.sync_copy(stage_v.at[pl.ds(0, 16)], out_hbm.at[pl.ds(_N - 16, 16)])


def _deg_call(ei, ones, zeros1d):
    k = pl.kernel(
        _deg_body,
        out_type=(jax.ShapeDtypeStruct((_N,), jnp.float32),
                  jax.ShapeDtypeStruct((_N,), jnp.float32)),
        mesh=_sc_mesh(),
        compiler_params=pltpu.CompilerParams(use_tc_tiling_on_sc=False),
        scratch_types=[
            pltpu.VMEM_SHARED((_N,), jnp.float32),
            pltpu.VMEM((_CH,), jnp.float32),
            pltpu.VMEM((_EPW,), jnp.int32),
            pltpu.VMEM((_RPT,), jnp.float32),
        ] + [pltpu.SemaphoreType.DMA] * _NBUF,
    )
    return k(ei, ones, zeros1d)


def _scat_body(ei_hbm, hs_hbm, z_hbm, out0_hbm, out1_hbm, acc,
               table, sidx, didx, stage_v, *bufs_and_sems, F):
    rows = bufs_and_sems[:_NBS]
    sems = bufs_and_sems[_NBS:]
    cid, sid, wid = _worker_ids()

    base_r = sid * _RPT
    pltpu.sync_copy(ei_hbm.at[0, pl.ds(wid * _EPW, _EPW)], sidx)
    pltpu.sync_copy(ei_hbm.at[1, pl.ds(wid * _EPW, _EPW)], didx)
    pltpu.sync_copy(z_hbm, stage_v)
    for p in range(_RPT // _SR):
        pltpu.sync_copy(stage_v, acc.at[pl.ds(base_r + p * _SR, _SR)])

    @pl.when(sid == _NS - 1)
    def _():
        pltpu.sync_copy(stage_v.at[pl.ds(0, 16)], acc.at[pl.ds(_N - 16, 16)])

    for p in range(_RPT // _SR):
        pltpu.sync_copy(hs_hbm.at[pl.ds(base_r + p * _SR, _SR)], stage_v)
        pltpu.sync_copy(stage_v, table.at[pl.ds(base_r + p * _SR, _SR)])

    @pl.when(sid == _NS - 1)
    def _():
        pltpu.sync_copy(hs_hbm.at[pl.ds(_N - 16, 16)], stage_v.at[pl.ds(0, 16)])
        pltpu.sync_copy(stage_v.at[pl.ds(0, 16)], table.at[pl.ds(_N - 16, 16)])

    plsc.subcore_barrier()

    gsem = sems[:_NBS]
    ssem = sems[_NBS:]

    def start_gather(i, b):
        pltpu.async_copy(table.at[sidx.at[pl.ds(i * _CH, _CH)]], rows[b], gsem[b])

    def wait_gather(b):
        pltpu.make_async_copy(table.at[sidx.at[pl.ds(0, _CH)]], rows[b], gsem[b]).wait()

    def start_scatter(i, b):
        pltpu.async_copy(rows[b], acc.at[didx.at[pl.ds(i * _CH, _CH)]], ssem[b],
                         add=True)

    def wait_scatter(b):
        pltpu.make_async_copy(rows[b], acc.at[didx.at[pl.ds(0, _CH)]], ssem[b]).wait()

    for b in range(_NBS):
        start_gather(b, b)
    for k in (0, 1):
        wait_gather(k)
        start_scatter(k, k)

    def group(g, carry):
        for r in range(_NBS):
            k = g * _NBS + 2 + r
            b = (2 + r) % _NBS
            wait_gather(b)
            start_scatter(k, b)
            wait_scatter(r)
            start_gather(k + 2, r)
        return carry

    lax.fori_loop(0, (_NCHUNK - 5) // _NBS, group, 0)
    wait_gather(2)
    start_scatter(122, 2)
    wait_scatter(0)
    start_gather(124, 0)
    wait_gather(3)
    start_scatter(123, 3)
    wait_gather(0)
    start_scatter(124, 0)
    for b in (1, 2, 3, 0):
        wait_scatter(b)

    plsc.subcore_barrier()

    for c, out_hbm in ((0, out0_hbm), (1, out1_hbm)):
        @pl.when(cid == c)
        def _():
            for p in range(_RPT // _SR):
                pltpu.sync_copy(acc.at[pl.ds(base_r + p * _SR, _SR)], stage_v)
                pltpu.sync_copy(stage_v, out_hbm.at[pl.ds(base_r + p * _SR, _SR)])

    @pl.when(sid == _NS - 1)
    def _():
        pltpu.sync_copy(acc.at[pl.ds(_N - 16, 16)], stage_v.at[pl.ds(0, 16)])
        for c, out_hbm in ((0, out0_hbm), (1, out1_hbm)):
            @pl.when(cid == c)
            def _():
                pltpu.sync_copy(stage_v.at[pl.ds(0, 16)], out_hbm.at[pl.ds(_N - 16, 16)])


def _scat_call(ei, hs, zeros2d, F):
    k = pl.kernel(
        functools.partial(_scat_body, F=F),
        out_type=(jax.ShapeDtypeStruct((_N, F), jnp.float32),
                  jax.ShapeDtypeStruct((_N, F), jnp.float32)),
        mesh=_sc_mesh(),
        compiler_params=pltpu.CompilerParams(use_tc_tiling_on_sc=False),
        scratch_types=[
            pltpu.VMEM_SHARED((_N, F), jnp.float32),
            pltpu.VMEM_SHARED((_N, F), jnp.float32),
            pltpu.VMEM((_EPW,), jnp.int32),
            pltpu.VMEM((_EPW,), jnp.int32),
            pltpu.VMEM((_SR, F), jnp.float32),
        ] + [pltpu.VMEM((_CH, F), jnp.float32)] * _NBS
          + [pltpu.SemaphoreType.DMA] * (2 * _NBS),
    )
    return k(ei, hs, zeros2d)


def _tc1_body(c0_ref, c1_ref, x_ref, w1_ref, hs1_ref, dinv_ref):
    c = c0_ref[...] + c1_ref[...]
    dinv = jnp.transpose(lax.rsqrt(c + 1.0), (1, 0))
    h = jnp.dot(x_ref[...], w1_ref[...], preferred_element_type=jnp.float32)
    hs1_ref[...] = h * dinv
    dinv_ref[...] = dinv


def _tc1_call(c0, c1, x, W1):
    return pl.pallas_call(
        _tc1_body,
        out_shape=[
            jax.ShapeDtypeStruct((_N, _H), jnp.float32),
            jax.ShapeDtypeStruct((_N, 1), jnp.float32),
        ],
    )(c0, c1, x, W1)


def _tc2_body(n1a_ref, n1b_ref, hs1_ref, dinv_ref, b1_ref, w2_ref, hs2_ref):
    n = n1a_ref[...] + n1b_ref[...] + hs1_ref[...]
    dinv = dinv_ref[...]
    out1 = jax.nn.relu(dinv * n + b1_ref[...])
    h2 = jnp.dot(out1, w2_ref[...], preferred_element_type=jnp.float32)
    hs2_ref[...] = jnp.concatenate(
        [h2 * dinv, jnp.zeros((_N, _CP - _C), jnp.float32)], axis=1)


def _tc2_call(n1a, n1b, hs1, dinv, b1r, W2):
    return pl.pallas_call(
        _tc2_body,
        out_shape=[
            jax.ShapeDtypeStruct((_N, _CP), jnp.float32),
        ],
    )(n1a, n1b, hs1, dinv, b1r, W2)


def _tc3_body(n2a_ref, n2b_ref, hs2_ref, dinv_ref, b2_ref, lsm_ref, xo_ref):
    n = (n2a_ref[...] + n2b_ref[...] + hs2_ref[...])[:, : _C]
    dinv = dinv_ref[...]
    xo = dinv * n + b2_ref[...]
    m = jnp.max(xo, axis=1, keepdims=True)
    lse = jnp.log(jnp.sum(jnp.exp(xo - m), axis=1, keepdims=True)) + m
    lsm_ref[...] = xo - lse
    xo_ref[...] = xo


def _tc3_call(n2a, n2b, hs2, dinv, b2r):
    return pl.pallas_call(
        _tc3_body,
        out_shape=[
            jax.ShapeDtypeStruct((_N, _C), jnp.float32),
            jax.ShapeDtypeStruct((_N, _C), jnp.float32),
        ],
    )(n2a, n2b, hs2, dinv, b2r)


def kernel(x, edge_index, W1, b1, W2, b2):
    ones = jnp.ones((_CH,), jnp.float32)
    zeros1d = jnp.zeros((_RPT,), jnp.float32)
    zerosH = jnp.zeros((_SR, _H), jnp.float32)
    zerosP = jnp.zeros((_SR, _CP), jnp.float32)

    c0, c1 = _deg_call(edge_index, ones, zeros1d)
    hs1, dinv = _tc1_call(c0.reshape(1, _N), c1.reshape(1, _N), x, W1)
    n1a, n1b = _scat_call(edge_index, hs1, zerosH, _H)
    hs2 = _tc2_call(n1a, n1b, hs1, dinv, b1.reshape(1, _H), W2)[0]
    n2a, n2b = _scat_call(edge_index, hs2, zerosP, _CP)
    lsm, xo = _tc3_call(n2a, n2b, hs2, dinv, b2.reshape(1, _C))
    return lsm, xo
